# baseline (speedup 1.0000x reference)
import functools

import jax
import jax.numpy as jnp
from jax import lax
from jax.experimental import pallas as pl
from jax.experimental.pallas import tpu as pltpu
from jax.experimental.pallas import tpu_sc as plsc

NUM_WORKERS = 32
LANES = 16


@jax.jit
def kernel(x, table):
    B, S = x.shape
    V, D = table.shape
    n_sblk = S // 8
    n_bblk = B // 128
    n_groups = S
    row_elems = (D // 8) * n_bblk * 8 * 128

    xn = x.astype(jnp.int32).reshape(n_bblk, 128, n_sblk, 8).transpose(2, 0, 3, 1)

    mesh = plsc.VectorSubcoreMesh(core_axis_name="c", subcore_axis_name="s")

    @functools.partial(
        pl.kernel,
        mesh=mesh,
        out_type=jax.ShapeDtypeStruct((S, row_elems), jnp.float32),
        compiler_params=pltpu.CompilerParams(
            use_tc_tiling_on_sc=False, needs_layout_passes=False
        ),
        scratch_types=[
            pltpu.VMEM((n_sblk, 8, 128), jnp.int32),
            pltpu.VMEM((128, D), jnp.float32),
            pltpu.VMEM((128, D), jnp.float32),
            pltpu.VMEM((128 * D,), jnp.float32),
            pltpu.VMEM((128 * D,), jnp.float32),
            pltpu.SemaphoreType.DMA,
            pltpu.SemaphoreType.DMA,
            pltpu.SemaphoreType.DMA,
            pltpu.SemaphoreType.DMA,
        ],
    )
    def emb(table_hbm, idx_hbm, out_hbm, idx_v, gbuf0, gbuf1, tbuf0, tbuf1,
            gsem0, gsem1, wsem0, wsem1):
        w = lax.axis_index("s") * 2 + lax.axis_index("c")
        pltpu.sync_copy(idx_hbm.at[:, w], idx_v)

        iota = lax.iota(jnp.int32, LANES)
        addr_h = [iota * 128 + 2048 * h for h in range(D // LANES)]

        def transpose_tile(gbuf, tbuf):
            for b in range(128):
                for h in range(D // LANES):
                    v = gbuf[b, pl.ds(h * LANES, LANES)]
                    plsc.store_scatter(tbuf, [addr_h[h] + b], v)

        def idx_slice(g):
            return idx_v.at[g // 8, g % 8]

        def fire_gather(g, gbuf, gsem):
            pltpu.async_copy(table_hbm.at[idx_slice(g)], gbuf, gsem)

        def out_chunk(g, tr):
            return out_hbm.at[g, pl.ds(tr * (n_bblk * 1024) + w * 1024, 1024)]

        def tbuf_chunk(tbuf, tr):
            return tbuf.at[pl.ds(tr * 1024, 1024)]

        def process(g, gbuf, tbuf, other_gbuf, gsem, other_gsem, wsem):
            pltpu.make_async_copy(table_hbm.at[idx_slice(g)], gbuf, gsem).wait()

            @pl.when(g + 1 < n_groups)
            def _():
                fire_gather(g + 1, other_gbuf, other_gsem)

            @pl.when(g >= 2)
            def _():
                for tr in range(D // 8):
                    pltpu.make_async_copy(
                        tbuf_chunk(tbuf, tr), out_chunk(g - 2, tr), wsem
                    ).wait()

            transpose_tile(gbuf, tbuf)
            for tr in range(D // 8):
                pltpu.async_copy(tbuf_chunk(tbuf, tr), out_chunk(g, tr), wsem)

        fire_gather(0, gbuf0, gsem0)

        def body(p, carry):
            process(2 * p, gbuf0, tbuf0, gbuf1, gsem0, gsem1, wsem0)
            process(2 * p + 1, gbuf1, tbuf1, gbuf0, gsem1, gsem0, wsem1)
            return carry

        lax.fori_loop(0, n_groups // 2, body, 0)

        for tr in range(D // 8):
            pltpu.make_async_copy(
                tbuf_chunk(tbuf0, tr), out_chunk(n_groups - 2, tr), wsem0
            ).wait()
            pltpu.make_async_copy(
                tbuf_chunk(tbuf1, tr), out_chunk(n_groups - 1, tr), wsem1
            ).wait()

    out2 = emb(table, xn)
    out5 = out2.reshape(S, D // 8, n_bblk, 8, 128)
    return out5.transpose(2, 4, 0, 1, 3).reshape(B, S, D)


# speedup vs baseline: 1.0107x; 1.0107x over previous
import functools

import jax
import jax.numpy as jnp
from jax import lax
from jax.experimental import pallas as pl
from jax.experimental.pallas import tpu as pltpu
from jax.experimental.pallas import tpu_sc as plsc

NUM_WORKERS = 32
NBUF = 4


@jax.jit
def kernel(x, table):
    B, S = x.shape
    V, D = table.shape
    n_sblk = S // 8
    n_bblk = B // 128

    xn = x.astype(jnp.int32).reshape(n_bblk, 128, n_sblk, 8).transpose(2, 0, 3, 1)

    mesh = plsc.VectorSubcoreMesh(core_axis_name="c", subcore_axis_name="s")

    @functools.partial(
        pl.kernel,
        mesh=mesh,
        out_type=jax.ShapeDtypeStruct((B, S, D), jnp.float32),
        compiler_params=pltpu.CompilerParams(use_tc_tiling_on_sc=False),
        scratch_types=[
            pltpu.VMEM((n_sblk, 8, 128), jnp.int32),
        ]
        + [pltpu.VMEM((128, D), jnp.float32) for _ in range(NBUF)]
        + [pltpu.SemaphoreType.DMA for _ in range(2 * NBUF)],
    )
    def emb(table_hbm, idx_hbm, out_hbm, idx_v, *bufs_and_sems):
        bufs = bufs_and_sems[:NBUF]
        gsems = bufs_and_sems[NBUF:2 * NBUF]
        wsems = bufs_and_sems[2 * NBUF:]
        w = lax.axis_index("s") * 2 + lax.axis_index("c")
        b0 = w * 128
        pltpu.sync_copy(idx_hbm.at[:, w], idx_v)

        def idx_slice(g):
            return idx_v.at[g // 8, g % 8]

        def out_slice(g):
            return out_hbm.at[pl.ds(b0, 128), g]

        def when(cond, fn):
            if isinstance(cond, bool):
                if cond:
                    fn()
            else:
                pl.when(cond)(fn)

        def fire_gather(g, j):
            pltpu.async_copy(table_hbm.at[idx_slice(g)], bufs[j], gsems[j])

        def process(g, j):
            pltpu.make_async_copy(
                table_hbm.at[idx_slice(g)], bufs[j], gsems[j]
            ).wait()

            jo = (j - 1) % NBUF
            when(g >= 1,
                 lambda: pltpu.make_async_copy(
                     bufs[jo], out_slice(g - 1), wsems[jo]).wait())
            when(g + 3 < S, lambda: fire_gather(g + 3, (j + 3) % NBUF))

            pltpu.async_copy(bufs[j], out_slice(g), wsems[j])

        for g in range(NBUF - 1):
            fire_gather(g, g)

        def body(p, carry):
            for j in range(NBUF):
                process(NBUF * p + j, j)
            return carry

        lax.fori_loop(0, S // NBUF, body, 0)

        j = (S - 1) % NBUF
        pltpu.make_async_copy(bufs[j], out_slice(S - 1), wsems[j]).wait()

    return emb(table, xn)


# speedup vs baseline: 1.0160x; 1.0052x over previous
import functools

import jax
import jax.numpy as jnp
from jax import lax
from jax.experimental import pallas as pl
from jax.experimental.pallas import tpu as pltpu
from jax.experimental.pallas import tpu_sc as plsc

NUM_WORKERS = 32
GROUP = 1024
K = 1


@jax.jit
def kernel(x, table):
    B, S = x.shape
    V, D = table.shape
    N = B * S
    n_per_w = N // NUM_WORKERS
    n_groups = n_per_w // GROUP
    n_blocks = n_groups // K

    idx = x.reshape(NUM_WORKERS, n_groups, GROUP).astype(jnp.int32)

    mesh = plsc.VectorSubcoreMesh(core_axis_name="c", subcore_axis_name="s")

    @functools.partial(
        pl.kernel,
        mesh=mesh,
        out_type=jax.ShapeDtypeStruct((N, D), jnp.float32),
        compiler_params=pltpu.CompilerParams(use_tc_tiling_on_sc=False),
        scratch_types=[
            pltpu.VMEM((n_groups, GROUP), jnp.int32),
            pltpu.VMEM((2 * K, GROUP, D), jnp.float32),
            pltpu.SemaphoreType.DMA,
            pltpu.SemaphoreType.DMA,
        ],
    )
    def emb(table_hbm, idx_hbm, out_hbm, idx_v, buf, gsem, wsem):
        wid = lax.axis_index("s") * 2 + lax.axis_index("c")
        base = wid * n_per_w
        pltpu.sync_copy(idx_hbm.at[wid], idx_v)

        for b in range(K):
            pltpu.async_copy(table_hbm.at[idx_v.at[b]], buf.at[b], gsem)

        def body(blk, carry):
            bank = (blk % 2) * K
            obank = ((blk + 1) % 2) * K

            for b in range(K):
                pltpu.make_async_copy(
                    table_hbm.at[idx_v.at[blk * K + b]], buf.at[bank + b], gsem
                ).wait()

            @pl.when(blk >= 1)
            def _():
                for b in range(K):
                    g = (blk - 1) * K + b
                    pltpu.make_async_copy(
                        buf.at[obank + b],
                        out_hbm.at[pl.ds(base + g * GROUP, GROUP)],
                        wsem,
                    ).wait()

            @pl.when(blk + 1 < n_blocks)
            def _():
                for b in range(K):
                    g = (blk + 1) * K + b
                    pltpu.async_copy(
                        table_hbm.at[idx_v.at[g]], buf.at[obank + b], gsem
                    )

            for b in range(K):
                g = blk * K + b
                pltpu.async_copy(
                    buf.at[bank + b],
                    out_hbm.at[pl.ds(base + g * GROUP, GROUP)],
                    wsem,
                )
            return carry

        lax.fori_loop(0, n_blocks, body, 0)

        last_bank = ((n_blocks - 1) % 2) * K
        for b in range(K):
            g = (n_blocks - 1) * K + b
            pltpu.make_async_copy(
                buf.at[last_bank + b],
                out_hbm.at[pl.ds(base + g * GROUP, GROUP)],
                wsem,
            ).wait()

    out = emb(table, idx)
    return out.reshape(B, S, D)
